# Initial kernel scaffold; baseline (speedup 1.0000x reference)
import jax
import jax.numpy as jnp
from jax.experimental import pallas as pl


def kernel(tokens, table):
    raise NotImplementedError("write your pallas kernel here")


# speedup vs baseline: 3.6787x; 3.6787x over previous
import functools

import jax
import jax.numpy as jnp
from jax import lax
from jax.experimental import pallas as pl
from jax.experimental.pallas import tpu as pltpu
from jax.experimental.pallas import tpu_sc as plsc

EMBED = 64
SCALE = 8.0
NC = 2
NS = 16
NW = NC * NS
GROUP = 128
NBUF = 4


def _scale_table(table):
    blk = 1024
    rows = table.shape[0]
    grid = (rows + blk - 1) // blk

    def body(t_ref, o_ref):
        o_ref[...] = t_ref[...] * SCALE

    return pl.pallas_call(
        body,
        out_shape=jax.ShapeDtypeStruct(table.shape, table.dtype),
        grid=(grid,),
        in_specs=[pl.BlockSpec((blk, EMBED), lambda i: (i, 0))],
        out_specs=pl.BlockSpec((blk, EMBED), lambda i: (i, 0)),
    )(table)


@functools.lru_cache(maxsize=None)
def _make_gather(num_tokens):
    assert num_tokens % (NW * GROUP) == 0
    g_per_w = num_tokens // (NW * GROUP)
    assert g_per_w % NBUF == 0
    steps = g_per_w // NBUF
    mesh = plsc.VectorSubcoreMesh(core_axis_name="c", subcore_axis_name="s")

    @functools.partial(
        pl.kernel,
        mesh=mesh,
        out_type=jax.ShapeDtypeStruct((num_tokens, EMBED), jnp.float32),
        scratch_types=(
            [pltpu.VMEM((g_per_w, GROUP), jnp.int32)]
            + [pltpu.VMEM((GROUP, EMBED), jnp.float32) for _ in range(NBUF)]
            + [pltpu.SemaphoreType.DMA for _ in range(2 * NBUF)]
        ),
        compiler_params=pltpu.CompilerParams(use_tc_tiling_on_sc=False),
    )
    def gather(tok_hbm, tab_hbm, out_hbm, idx_v, *rest):
        rows = rest[:NBUF]
        sg = rest[NBUF:2 * NBUF]
        sw = rest[2 * NBUF:]
        wid = lax.axis_index("s") * NC + lax.axis_index("c")
        base = wid * (g_per_w * GROUP)
        pltpu.sync_copy(tok_hbm.at[wid], idx_v)

        def step(p, carry):
            g0 = p * NBUF
            for b in range(NBUF):
                pltpu.make_async_copy(
                    tab_hbm.at[idx_v.at[g0 + b]], rows[b], sg[b]).start()
            for b in range(NBUF):
                pltpu.make_async_copy(
                    tab_hbm.at[idx_v.at[g0 + b]], rows[b], sg[b]).wait()
                pltpu.make_async_copy(
                    rows[b],
                    out_hbm.at[pl.ds(base + (g0 + b) * GROUP, GROUP)],
                    sw[b]).start()
            for b in range(NBUF):
                pltpu.make_async_copy(
                    rows[b],
                    out_hbm.at[pl.ds(base + (g0 + b) * GROUP, GROUP)],
                    sw[b]).wait()
            return carry

        lax.fori_loop(0, steps, step, 0)

    return gather


def kernel(tokens, table):
    batch, seq = tokens.shape
    num_tokens = batch * seq
    idx = tokens.astype(jnp.int32).reshape(NW, num_tokens // (NW * GROUP), GROUP)
    scaled = _scale_table(table)
    out = _make_gather(num_tokens)(idx, scaled)
    return out.reshape(batch, seq, EMBED)


# speedup vs baseline: 3.8326x; 1.0418x over previous
import functools

import jax
import jax.numpy as jnp
from jax import lax
from jax.experimental import pallas as pl
from jax.experimental.pallas import tpu as pltpu
from jax.experimental.pallas import tpu_sc as plsc

EMBED = 64
SCALE = 8.0
NC = 2
NS = 16
NW = NC * NS
GROUP = 128
NBUF = 4


@functools.lru_cache(maxsize=None)
def _make_gather(num_tokens):
    assert num_tokens % (NW * GROUP) == 0
    g_per_w = num_tokens // (NW * GROUP)
    assert g_per_w % NBUF == 0
    steps = g_per_w // NBUF
    mesh = plsc.VectorSubcoreMesh(core_axis_name="c", subcore_axis_name="s")

    @functools.partial(
        pl.kernel,
        mesh=mesh,
        out_type=jax.ShapeDtypeStruct((num_tokens, EMBED), jnp.float32),
        scratch_types=(
            [pltpu.VMEM((g_per_w, GROUP), jnp.int32)]
            + [pltpu.VMEM((GROUP, EMBED), jnp.float32) for _ in range(NBUF)]
            + [pltpu.SemaphoreType.DMA for _ in range(2 * NBUF)]
        ),
        compiler_params=pltpu.CompilerParams(use_tc_tiling_on_sc=False),
    )
    def gather(tok_hbm, tab_hbm, out_hbm, idx_v, *rest):
        rows = rest[:NBUF]
        sg = rest[NBUF:2 * NBUF]
        sw = rest[2 * NBUF:]
        wid = lax.axis_index("s") * NC + lax.axis_index("c")
        base = wid * (g_per_w * GROUP)
        pltpu.sync_copy(tok_hbm.at[wid], idx_v)

        def step(p, carry):
            g0 = p * NBUF
            for b in range(NBUF):
                pltpu.make_async_copy(
                    tab_hbm.at[idx_v.at[g0 + b]], rows[b], sg[b]).start()
            for b in range(NBUF):
                pltpu.make_async_copy(
                    tab_hbm.at[idx_v.at[g0 + b]], rows[b], sg[b]).wait()
                pltpu.make_async_copy(
                    rows[b],
                    out_hbm.at[pl.ds(base + (g0 + b) * GROUP, GROUP)],
                    sw[b]).start()
            for b in range(NBUF):
                pltpu.make_async_copy(
                    rows[b],
                    out_hbm.at[pl.ds(base + (g0 + b) * GROUP, GROUP)],
                    sw[b]).wait()
            return carry

        lax.fori_loop(0, steps, step, 0)

    return gather


def _unpack_scale(packed, num_tokens):
    rows_in = 1600

    def body(p_ref, o_ref):
        x = p_ref[...]
        xx = jnp.broadcast_to(x[:, None, :], (rows_in, 2, 2 * EMBED))
        xx = xx.reshape(2 * rows_in, 2 * EMBED)
        par = jax.lax.broadcasted_iota(jnp.int32, (2 * rows_in, 1), 0) & 1
        y = jnp.where(par == 0, xx[:, :EMBED], xx[:, EMBED:])
        o_ref[...] = y * SCALE

    return pl.pallas_call(
        body,
        out_shape=jax.ShapeDtypeStruct((num_tokens, EMBED), jnp.float32),
        grid=(num_tokens // (2 * rows_in),),
        in_specs=[pl.BlockSpec((rows_in, 2 * EMBED), lambda i: (i, 0))],
        out_specs=pl.BlockSpec((2 * rows_in, EMBED), lambda i: (i, 0)),
    )(packed)


def kernel(tokens, table):
    batch, seq = tokens.shape
    num_tokens = batch * seq
    idx = tokens.astype(jnp.int32).reshape(NW, num_tokens // (NW * GROUP), GROUP)
    gathered = _make_gather(num_tokens)(idx, table)
    packed = gathered.reshape(num_tokens // 2, 2 * EMBED)
    return _unpack_scale(packed, num_tokens).reshape(batch, seq, EMBED)


# speedup vs baseline: 4.2847x; 1.1180x over previous
import functools

import jax
import jax.numpy as jnp
from jax import lax
from jax.experimental import pallas as pl
from jax.experimental.pallas import tpu as pltpu
from jax.experimental.pallas import tpu_sc as plsc

EMBED = 64
SCALE = 8.0
NC = 2
NS = 16
NW = NC * NS
GROUP = 128
NBUF = 4


@functools.lru_cache(maxsize=None)
def _make_gather(num_tokens):
    assert num_tokens % (NW * GROUP) == 0
    g_per_w = num_tokens // (NW * GROUP)
    assert g_per_w % NBUF == 0
    steps = g_per_w // NBUF
    mesh = plsc.VectorSubcoreMesh(core_axis_name="c", subcore_axis_name="s")

    @functools.partial(
        pl.kernel,
        mesh=mesh,
        out_type=jax.ShapeDtypeStruct((num_tokens, EMBED), jnp.float32),
        scratch_types=(
            [pltpu.VMEM((g_per_w, GROUP), jnp.int32)]
            + [pltpu.VMEM((GROUP, EMBED), jnp.float32) for _ in range(NBUF)]
            + [pltpu.SemaphoreType.DMA for _ in range(2 * NBUF)]
        ),
        compiler_params=pltpu.CompilerParams(use_tc_tiling_on_sc=False),
    )
    def gather(tok_hbm, tab_hbm, out_hbm, idx_v, *rest):
        rows = rest[:NBUF]
        sg = rest[NBUF:2 * NBUF]
        sw = rest[2 * NBUF:]
        wid = lax.axis_index("s") * NC + lax.axis_index("c")
        base = wid * (g_per_w * GROUP)
        pltpu.sync_copy(tok_hbm.at[wid], idx_v)

        def step(p, carry):
            g0 = p * NBUF
            for b in range(NBUF):
                pltpu.make_async_copy(
                    tab_hbm.at[idx_v.at[g0 + b]], rows[b], sg[b]).start()
            for b in range(NBUF):
                pltpu.make_async_copy(
                    tab_hbm.at[idx_v.at[g0 + b]], rows[b], sg[b]).wait()
                pltpu.make_async_copy(
                    rows[b],
                    out_hbm.at[pl.ds(base + (g0 + b) * GROUP, GROUP)],
                    sw[b]).start()
            for b in range(NBUF):
                pltpu.make_async_copy(
                    rows[b],
                    out_hbm.at[pl.ds(base + (g0 + b) * GROUP, GROUP)],
                    sw[b]).wait()
            return carry

        lax.fori_loop(0, steps, step, 0)

    return gather


def _unpack_scale(packed, num_tokens):
    rows_in = 1600

    def body(p_ref, o_ref):
        x = p_ref[...] * SCALE
        o_ref[pl.Slice(0, rows_in, 2), :] = x[:, :EMBED]
        o_ref[pl.Slice(1, rows_in, 2), :] = x[:, EMBED:]

    return pl.pallas_call(
        body,
        out_shape=jax.ShapeDtypeStruct((num_tokens, EMBED), jnp.float32),
        grid=(num_tokens // (2 * rows_in),),
        in_specs=[pl.BlockSpec((rows_in, 2 * EMBED), lambda i: (i, 0))],
        out_specs=pl.BlockSpec((2 * rows_in, EMBED), lambda i: (i, 0)),
    )(packed)


def kernel(tokens, table):
    batch, seq = tokens.shape
    num_tokens = batch * seq
    idx = tokens.astype(jnp.int32).reshape(NW, num_tokens // (NW * GROUP), GROUP)
    gathered = _make_gather(num_tokens)(idx, table)
    packed = gathered.reshape(num_tokens // 2, 2 * EMBED)
    return _unpack_scale(packed, num_tokens).reshape(batch, seq, EMBED)
